# Initial kernel scaffold; baseline (speedup 1.0000x reference)
import jax
import jax.numpy as jnp
from jax.experimental import pallas as pl


def kernel(sigma, sigmas):
    raise NotImplementedError("write your pallas kernel here")


# speedup vs baseline: 49.8219x; 49.8219x over previous
import functools

import jax
import jax.numpy as jnp
from jax import lax
from jax.experimental import pallas as pl
from jax.experimental.pallas import tpu as pltpu
from jax.experimental.pallas import tpu_sc as plsc

NC = 2
NS = 16
NW = NC * NS
L = 16
B = 65536
BPW = B // NW
NSIG = 1000
NPAD = 1024
BIG = 3.0e37
SEARCH_STEPS = 10


def _sigma_to_t_body(sigma_hbm, sigmas_hbm, out_hbm, tab_v, q_v, o_v):
    wid = lax.axis_index("s") * NC + lax.axis_index("c")
    base = wid * BPW
    pltpu.sync_copy(sigmas_hbm, tab_v)
    pltpu.sync_copy(sigma_hbm.at[pl.ds(base, BPW)], q_v)

    def step(i, carry):
        q = q_v[pl.ds(i * L, L)]
        lo = jnp.zeros((L,), jnp.int32)
        hi = jnp.full((L,), NSIG, jnp.int32)
        for _ in range(SEARCH_STEPS):
            mid = (lo + hi) >> 1
            v = plsc.load_gather(tab_v, [mid])
            pred = v < q
            lo = jnp.where(pred, mid + 1, lo)
            hi = jnp.where(pred, hi, mid)
        c = lo

        ib1 = jnp.maximum(c - 1, 0)
        ib2 = jnp.maximum(c - 2, 0)
        ia1 = jnp.minimum(c, NSIG - 1)
        ia2 = jnp.minimum(c + 1, NSIG - 1)
        vb1 = plsc.load_gather(tab_v, [ib1])
        vb2 = plsc.load_gather(tab_v, [ib2])
        va1 = plsc.load_gather(tab_v, [ia1])
        va2 = plsc.load_gather(tab_v, [ia2])
        db1 = jnp.where(c >= 1, q - vb1, BIG)
        db2 = jnp.where(c >= 2, q - vb2, BIG)
        da1 = jnp.where(c <= NSIG - 1, va1 - q, BIG)
        da2 = jnp.where(c <= NSIG - 2, va2 - q, BIG)

        nb = db1 <= da1
        i_n = jnp.where(nb, c - 1, c)
        v_n = jnp.where(nb, vb1, va1)
        sec_b = db2 <= da1
        sec_a = db1 <= da2
        i_s = jnp.where(nb, jnp.where(sec_b, c - 2, c),
                        jnp.where(sec_a, c - 1, c + 1))
        v_s = jnp.where(nb, jnp.where(sec_b, vb2, va1),
                        jnp.where(sec_a, vb1, va2))

        lo_first = i_n < i_s
        low_i = jnp.where(lo_first, i_n, i_s)
        high_i = jnp.where(lo_first, i_s, i_n)
        low_v = jnp.where(lo_first, v_n, v_s)
        high_v = jnp.where(lo_first, v_s, v_n)

        w = jnp.clip((low_v - q) / (low_v - high_v), 0.0, 1.0)
        t = (1.0 - w) * low_i.astype(jnp.float32) \
            + w * high_i.astype(jnp.float32)
        o_v[pl.ds(i * L, L)] = t
        return carry

    lax.fori_loop(0, BPW // L, step, 0)
    pltpu.sync_copy(o_v, out_hbm.at[pl.ds(base, BPW)])


@jax.jit
def kernel(sigma, sigmas):
    sigmas_padded = jnp.concatenate(
        [sigmas, jnp.full((NPAD - NSIG,), BIG, jnp.float32)])
    mesh = plsc.VectorSubcoreMesh(core_axis_name="c", subcore_axis_name="s")
    run = pl.kernel(
        _sigma_to_t_body,
        mesh=mesh,
        out_type=jax.ShapeDtypeStruct((B,), jnp.float32),
        scratch_types=[
            pltpu.VMEM((NPAD,), jnp.float32),
            pltpu.VMEM((BPW,), jnp.float32),
            pltpu.VMEM((BPW,), jnp.float32),
        ],
        compiler_params=pltpu.CompilerParams(needs_layout_passes=False),
    )
    return run(sigma, sigmas_padded)


# speedup vs baseline: 66.8180x; 1.3411x over previous
import functools

import jax
import jax.numpy as jnp
from jax import lax
from jax.experimental import pallas as pl
from jax.experimental.pallas import tpu as pltpu
from jax.experimental.pallas import tpu_sc as plsc

NC = 2
NS = 16
NW = NC * NS
L = 16
B = 65536
BPW = B // NW
NSIG = 1000
NPAD = 1024
BIG = 3.0e37
SEARCH_STEPS = 10


def _sigma_to_t_body(sigma_hbm, sigmas_hbm, out_hbm, tab_v, q_v, o_v):
    wid = lax.axis_index("s") * NC + lax.axis_index("c")
    base = wid * BPW
    pltpu.sync_copy(sigmas_hbm, tab_v)
    pltpu.sync_copy(sigma_hbm.at[pl.ds(base, BPW)], q_v)

    @plsc.parallel_loop(0, BPW // L, unroll=4)
    def step(i):
        q = q_v[pl.ds(i * L, L)]
        lo = jnp.zeros((L,), jnp.int32)
        hi = jnp.full((L,), NSIG, jnp.int32)
        for _ in range(SEARCH_STEPS):
            mid = (lo + hi) >> 1
            v = plsc.load_gather(tab_v, [mid])
            pred = v < q
            lo = jnp.where(pred, mid + 1, lo)
            hi = jnp.where(pred, hi, mid)
        c = lo

        ib1 = jnp.maximum(c - 1, 0)
        ib2 = jnp.maximum(c - 2, 0)
        ia1 = jnp.minimum(c, NSIG - 1)
        ia2 = jnp.minimum(c + 1, NSIG - 1)
        vb1 = plsc.load_gather(tab_v, [ib1])
        vb2 = plsc.load_gather(tab_v, [ib2])
        va1 = plsc.load_gather(tab_v, [ia1])
        va2 = plsc.load_gather(tab_v, [ia2])
        db1 = jnp.where(c >= 1, q - vb1, BIG)
        db2 = jnp.where(c >= 2, q - vb2, BIG)
        da1 = jnp.where(c <= NSIG - 1, va1 - q, BIG)
        da2 = jnp.where(c <= NSIG - 2, va2 - q, BIG)

        nb = db1 <= da1
        i_n = jnp.where(nb, c - 1, c)
        v_n = jnp.where(nb, vb1, va1)
        sec_b = db2 <= da1
        sec_a = db1 <= da2
        i_s = jnp.where(nb, jnp.where(sec_b, c - 2, c),
                        jnp.where(sec_a, c - 1, c + 1))
        v_s = jnp.where(nb, jnp.where(sec_b, vb2, va1),
                        jnp.where(sec_a, vb1, va2))

        lo_first = i_n < i_s
        low_i = jnp.where(lo_first, i_n, i_s)
        high_i = jnp.where(lo_first, i_s, i_n)
        low_v = jnp.where(lo_first, v_n, v_s)
        high_v = jnp.where(lo_first, v_s, v_n)

        w = jnp.clip((low_v - q) / (low_v - high_v), 0.0, 1.0)
        t = (1.0 - w) * low_i.astype(jnp.float32) \
            + w * high_i.astype(jnp.float32)
        o_v[pl.ds(i * L, L)] = t

    pltpu.sync_copy(o_v, out_hbm.at[pl.ds(base, BPW)])


@jax.jit
def kernel(sigma, sigmas):
    sigmas_padded = jnp.concatenate(
        [sigmas, jnp.full((NPAD - NSIG,), BIG, jnp.float32)])
    mesh = plsc.VectorSubcoreMesh(core_axis_name="c", subcore_axis_name="s")
    run = pl.kernel(
        _sigma_to_t_body,
        mesh=mesh,
        out_type=jax.ShapeDtypeStruct((B,), jnp.float32),
        scratch_types=[
            pltpu.VMEM((NPAD,), jnp.float32),
            pltpu.VMEM((BPW,), jnp.float32),
            pltpu.VMEM((BPW,), jnp.float32),
        ],
        compiler_params=pltpu.CompilerParams(needs_layout_passes=False),
    )
    return run(sigma, sigmas_padded)


# speedup vs baseline: 68.1904x; 1.0205x over previous
import functools

import jax
import jax.numpy as jnp
from jax import lax
from jax.experimental import pallas as pl
from jax.experimental.pallas import tpu as pltpu
from jax.experimental.pallas import tpu_sc as plsc

NC = 2
NS = 16
NW = NC * NS
L = 16
B = 65536
BPW = B // NW
NSIG = 1000
NPAD = 1024
BIG = 3.0e37
SEARCH_STEPS = 10


def _sigma_to_t_body(sigma_hbm, sigmas_hbm, out_hbm, tab_v, q_v, o_v):
    wid = lax.axis_index("s") * NC + lax.axis_index("c")
    base = wid * BPW
    pltpu.sync_copy(sigmas_hbm, tab_v)
    pltpu.sync_copy(sigma_hbm.at[pl.ds(base, BPW)], q_v)

    @plsc.parallel_loop(0, BPW // L, unroll=4)
    def step(i):
        q = q_v[pl.ds(i * L, L)]
        lo = jnp.zeros((L,), jnp.int32)
        hi = jnp.full((L,), NSIG, jnp.int32)
        for _ in range(SEARCH_STEPS):
            mid = (lo + hi) >> 1
            v = plsc.load_gather(tab_v, [jnp.minimum(mid, NSIG - 1)])
            pred = v < q
            lo = jnp.where(pred, mid + 1, lo)
            hi = jnp.where(pred, hi, mid)
        c = hi

        ib1 = jnp.maximum(c - 1, 0)
        ib2 = jnp.maximum(c - 2, 0)
        ia1 = jnp.minimum(c, NSIG - 1)
        ia2 = jnp.minimum(c + 1, NSIG - 1)
        vb1 = plsc.load_gather(tab_v, [ib1])
        vb2 = plsc.load_gather(tab_v, [ib2])
        va1 = plsc.load_gather(tab_v, [ia1])
        va2 = plsc.load_gather(tab_v, [ia2])
        db1 = jnp.where(c >= 1, q - vb1, BIG)
        db2 = jnp.where(c >= 2, q - vb2, BIG)
        da1 = jnp.where(c <= NSIG - 1, va1 - q, BIG)
        da2 = jnp.where(c <= NSIG - 2, va2 - q, BIG)

        nb = db1 <= da1
        i_n = jnp.where(nb, c - 1, c)
        v_n = jnp.where(nb, vb1, va1)
        sec_b = db2 <= da1
        sec_a = db1 <= da2
        i_s = jnp.where(nb, jnp.where(sec_b, c - 2, c),
                        jnp.where(sec_a, c - 1, c + 1))
        v_s = jnp.where(nb, jnp.where(sec_b, vb2, va1),
                        jnp.where(sec_a, vb1, va2))

        lo_first = i_n < i_s
        low_i = jnp.where(lo_first, i_n, i_s)
        high_i = jnp.where(lo_first, i_s, i_n)
        low_v = jnp.where(lo_first, v_n, v_s)
        high_v = jnp.where(lo_first, v_s, v_n)

        w = jnp.clip((low_v - q) / (low_v - high_v), 0.0, 1.0)
        t = (1.0 - w) * low_i.astype(jnp.float32) \
            + w * high_i.astype(jnp.float32)
        o_v[pl.ds(i * L, L)] = t

    pltpu.sync_copy(o_v, out_hbm.at[pl.ds(base, BPW)])


@jax.jit
def kernel(sigma, sigmas):
    mesh = plsc.VectorSubcoreMesh(core_axis_name="c", subcore_axis_name="s")
    run = pl.kernel(
        _sigma_to_t_body,
        mesh=mesh,
        out_type=jax.ShapeDtypeStruct((B,), jnp.float32),
        scratch_types=[
            pltpu.VMEM((NSIG,), jnp.float32),
            pltpu.VMEM((BPW,), jnp.float32),
            pltpu.VMEM((BPW,), jnp.float32),
        ],
        compiler_params=pltpu.CompilerParams(needs_layout_passes=False),
    )
    return run(sigma, sigmas)
